# Initial kernel scaffold; baseline (speedup 1.0000x reference)
import jax
import jax.numpy as jnp
from jax.experimental import pallas as pl


def kernel(word_ids, table):
    raise NotImplementedError("write your pallas kernel here")


# speedup vs baseline: 5.8016x; 5.8016x over previous
import functools

import jax
import jax.numpy as jnp
from jax import lax
from jax.experimental import pallas as pl
from jax.experimental.pallas import tpu as pltpu
from jax.experimental.pallas import tpu_sc as plsc


def _make_gather(N, V, D, NW, CHUNK):
    n_per_w = N // NW
    n_chunks = n_per_w // CHUNK
    mesh = plsc.VectorSubcoreMesh(core_axis_name="c", subcore_axis_name="s")

    @functools.partial(
        pl.kernel,
        out_type=jax.ShapeDtypeStruct((N, D), jnp.float32),
        mesh=mesh,
        scratch_types=[
            pltpu.VMEM((CHUNK,), jnp.int32),
            pltpu.VMEM((CHUNK, D), jnp.float32),
            pltpu.SemaphoreType.DMA,
        ],
        compiler_params=pltpu.CompilerParams(use_tc_tiling_on_sc=False),
    )
    def gather_kernel(ids_hbm, table_hbm, out_hbm, idx_v, rows_v, sem):
        wid = lax.axis_index("s") * 2 + lax.axis_index("c")
        base = wid * n_per_w

        @pl.loop(0, n_chunks)
        def _(i):
            off = base + i * CHUNK
            pltpu.sync_copy(ids_hbm.at[pl.ds(off, CHUNK)], idx_v)
            pltpu.async_copy(table_hbm.at[idx_v], rows_v, sem).wait()
            pltpu.sync_copy(rows_v, out_hbm.at[pl.ds(off, CHUNK)])

    return gather_kernel


def kernel(word_ids, table):
    B, H = word_ids.shape
    V, D = table.shape
    N = B * H
    NW = 32
    CHUNK = 512
    flat_ids = word_ids.reshape(N).astype(jnp.int32)
    out = _make_gather(N, V, D, NW, CHUNK)(flat_ids, table)
    return out.reshape(B, H, D)


# speedup vs baseline: 6.2154x; 1.0713x over previous
import functools

import jax
import jax.numpy as jnp
from jax import lax
from jax.experimental import pallas as pl
from jax.experimental.pallas import tpu as pltpu
from jax.experimental.pallas import tpu_sc as plsc


def _make_gather(N, V, D, NW, CHUNK):
    n_per_w = N // NW
    n_chunks = n_per_w // CHUNK
    assert n_chunks % 2 == 0 and n_chunks >= 6
    n_pairs = n_chunks // 2
    mesh = plsc.VectorSubcoreMesh(core_axis_name="c", subcore_axis_name="s")

    @functools.partial(
        pl.kernel,
        out_type=jax.ShapeDtypeStruct((N, D), jnp.float32),
        mesh=mesh,
        scratch_types=[
            pltpu.VMEM((CHUNK,), jnp.int32),
            pltpu.VMEM((CHUNK,), jnp.int32),
            pltpu.VMEM((CHUNK, D), jnp.float32),
            pltpu.VMEM((CHUNK, D), jnp.float32),
            pltpu.SemaphoreType.DMA,
            pltpu.SemaphoreType.DMA,
            pltpu.SemaphoreType.DMA,
            pltpu.SemaphoreType.DMA,
            pltpu.SemaphoreType.DMA,
            pltpu.SemaphoreType.DMA,
        ],
        compiler_params=pltpu.CompilerParams(use_tc_tiling_on_sc=False),
    )
    def gather_kernel(ids_hbm, table_hbm, out_hbm,
                      idx0, idx1, rows0, rows1,
                      si0, si1, sg0, sg1, so0, so1):
        idx_v = (idx0, idx1)
        rows_v = (rows0, rows1)
        si = (si0, si1)
        sg = (sg0, sg1)
        so = (so0, so1)
        wid = lax.axis_index("s") * 2 + lax.axis_index("c")
        base = wid * n_per_w

        def fire_idx(i, b):
            pltpu.async_copy(ids_hbm.at[pl.ds(base + i * CHUNK, CHUNK)],
                             idx_v[b], si[b])

        def fire_gather(b):
            pltpu.async_copy(table_hbm.at[idx_v[b]], rows_v[b], sg[b])

        def fire_out(i, b):
            pltpu.async_copy(rows_v[b],
                             out_hbm.at[pl.ds(base + i * CHUNK, CHUNK)],
                             so[b])

        def wait_idx(b):
            pltpu.make_async_copy(ids_hbm.at[pl.ds(0, CHUNK)],
                                  idx_v[b], si[b]).wait()

        def wait_gather(b):
            pltpu.make_async_copy(table_hbm.at[idx_v[b]],
                                  rows_v[b], sg[b]).wait()

        def wait_out(b):
            pltpu.make_async_copy(rows_v[b],
                                  out_hbm.at[pl.ds(0, CHUNK)], so[b]).wait()

        for b in range(2):
            fire_idx(b, b)
        for b in range(2):
            wait_idx(b)
            fire_gather(b)
            wait_gather(b)
            fire_out(b, b)
            fire_idx(b + 2, b)

        @pl.loop(1, n_pairs - 1)
        def _(p):
            for b in range(2):
                i = 2 * p + b
                wait_out(b)
                wait_idx(b)
                fire_gather(b)
                wait_gather(b)
                fire_out(i, b)
                fire_idx(i + 2, b)

        for b in range(2):
            i = n_chunks - 2 + b
            wait_out(b)
            wait_idx(b)
            fire_gather(b)
            wait_gather(b)
            fire_out(i, b)
        for b in range(2):
            wait_out(b)

    return gather_kernel


def kernel(word_ids, table):
    B, H = word_ids.shape
    V, D = table.shape
    N = B * H
    NW = 32
    CHUNK = 512
    flat_ids = word_ids.reshape(N).astype(jnp.int32)
    out = _make_gather(N, V, D, NW, CHUNK)(flat_ids, table)
    return out.reshape(B, H, D)


# speedup vs baseline: 6.2312x; 1.0025x over previous
import functools

import jax
import jax.numpy as jnp
from jax import lax
from jax.experimental import pallas as pl
from jax.experimental.pallas import tpu as pltpu
from jax.experimental.pallas import tpu_sc as plsc


def _make_gather(N, V, D, NW, CHUNK, NB, LAG):
    n_per_w = N // NW
    n_chunks = n_per_w // CHUNK
    assert n_per_w % CHUNK == 0 and n_chunks % NB == 0 and n_chunks >= 2 * NB
    assert LAG <= NB
    mesh = plsc.VectorSubcoreMesh(core_axis_name="c", subcore_axis_name="s")

    @functools.partial(
        pl.kernel,
        out_type=jax.ShapeDtypeStruct((N, D), jnp.float32),
        mesh=mesh,
        scratch_types=(
            [pltpu.VMEM((CHUNK,), jnp.int32) for _ in range(NB)]
            + [pltpu.VMEM((CHUNK, D), jnp.float32) for _ in range(NB)]
            + [pltpu.SemaphoreType.DMA for _ in range(3 * NB)]
        ),
        compiler_params=pltpu.CompilerParams(use_tc_tiling_on_sc=False),
    )
    def gather_kernel(ids_hbm, table_hbm, out_hbm, *scratch):
        idx_v = scratch[:NB]
        rows_v = scratch[NB:2 * NB]
        si = scratch[2 * NB:3 * NB]
        sg = scratch[3 * NB:4 * NB]
        so = scratch[4 * NB:5 * NB]
        wid = lax.axis_index("s") * 2 + lax.axis_index("c")
        base = wid * n_per_w

        def fire_idx(i, b):
            pltpu.async_copy(ids_hbm.at[pl.ds(base + i * CHUNK, CHUNK)],
                             idx_v[b], si[b])

        def fire_gather(b):
            pltpu.async_copy(table_hbm.at[idx_v[b]], rows_v[b], sg[b])

        def fire_out(i, b):
            pltpu.async_copy(rows_v[b],
                             out_hbm.at[pl.ds(base + i * CHUNK, CHUNK)],
                             so[b])

        def wait_idx(b):
            pltpu.make_async_copy(ids_hbm.at[pl.ds(0, CHUNK)],
                                  idx_v[b], si[b]).wait()

        def wait_gather(b):
            pltpu.make_async_copy(table_hbm.at[idx_v[b]],
                                  rows_v[b], sg[b]).wait()

        def wait_out(b):
            pltpu.make_async_copy(rows_v[b],
                                  out_hbm.at[pl.ds(0, CHUNK)], so[b]).wait()

        def complete(j, bj, prefetch):
            wait_gather(bj)
            fire_out(j, bj)
            if prefetch:
                fire_idx(j + NB, bj)

        for b in range(NB):
            fire_idx(b, b)
        for i in range(LAG):
            wait_idx(i)
            fire_gather(i)
        for i in range(LAG, NB):
            wait_idx(i)
            fire_gather(i)
            complete(i - LAG, i - LAG, True)

        @pl.loop(1, n_chunks // NB)
        def _(g):
            for b in range(NB):
                i = NB * g + b
                wait_out(b)
                wait_idx(b)
                fire_gather(b)
                j = i - LAG
                fire_idx_chunk = jnp.minimum(j + NB, n_chunks - 1)
                bj = (b - LAG) % NB
                wait_gather(bj)
                fire_out(j, bj)
                fire_idx(fire_idx_chunk, bj)

        for j in range(n_chunks - LAG, n_chunks):
            complete(j, j % NB, False)
        for b in range(NB - LAG):
            wait_idx(b)
        for b in range(NB):
            wait_out(b)

    return gather_kernel


def kernel(word_ids, table):
    B, H = word_ids.shape
    V, D = table.shape
    N = B * H
    NW = 32
    CHUNK = 200
    NB = 8
    LAG = 4
    flat_ids = word_ids.reshape(N).astype(jnp.int32)
    out = _make_gather(N, V, D, NW, CHUNK, NB, LAG)(flat_ids, table)
    return out.reshape(B, H, D)
